# baseline (speedup 1.0000x reference)
import jax
import jax.numpy as jnp
from jax.experimental import pallas as pl
from jax.experimental.pallas import tpu as pltpu

_NUM_TYPE = 1000
_BATCH = 1024
_HIST = 200
_TN = 200
_GRID = _NUM_TYPE // _TN
_LANES = 128
_SUB = _BATCH // _LANES


def _body(ti_hbm, tjs_hbm, h_hbm, emb_hbm, out_ref,
          gtT_ref, hf_ref, ebuf_ref, ti_ref, tjs_ref, h_ref, sems, ssems):
    i = pl.program_id(0)

    @pl.when(i == 0)
    def _init():
        for blk in range(_GRID):
            pltpu.make_async_copy(
                emb_hbm.at[pl.ds(blk * _TN, _TN), :], ebuf_ref.at[blk],
                sems.at[blk]).start()
        cp_ti = pltpu.make_async_copy(ti_hbm, ti_ref, ssems.at[0])
        cp_tjs = pltpu.make_async_copy(tjs_hbm, tjs_ref, ssems.at[1])
        cp_h = pltpu.make_async_copy(h_hbm, h_ref, ssems.at[2])
        cp_ti.start()
        cp_tjs.start()
        cp_h.start()
        cp_ti.wait()
        cp_tjs.wait()
        gtT_ref[:] = jnp.exp(tjs_ref[0, :][:, None] - ti_ref[0, :][None, :])
        cp_h.wait()
        hf_ref[:] = h_ref[0].astype(jnp.float32)

    pltpu.make_async_copy(
        emb_hbm.at[pl.ds(i * _TN, _TN), :], ebuf_ref.at[i],
        sems.at[i]).wait()

    P = jax.lax.dot_general(
        ebuf_ref[i], hf_ref[:], (((1,), (1,)), ((), ())),
        preferred_element_type=jnp.float32)
    res = jnp.dot(P, gtT_ref[:], preferred_element_type=jnp.float32)
    out_ref[:] = res.reshape(_TN, _SUB, _LANES)


def kernel(ti, tjs, ci, cjs, emb_weight):
    del ci
    ti_row = jnp.reshape(ti, (1, _BATCH))
    out = pl.pallas_call(
        _body,
        grid=(_GRID,),
        in_specs=[
            pl.BlockSpec(memory_space=pl.ANY),
            pl.BlockSpec(memory_space=pl.ANY),
            pl.BlockSpec(memory_space=pl.ANY),
            pl.BlockSpec(memory_space=pl.ANY),
        ],
        out_specs=pl.BlockSpec((_TN, _SUB, _LANES), lambda i: (i, 0, 0)),
        out_shape=jax.ShapeDtypeStruct((_NUM_TYPE, _SUB, _LANES), jnp.float32),
        scratch_shapes=[
            pltpu.VMEM((_HIST, _BATCH), jnp.float32),
            pltpu.VMEM((_HIST, _NUM_TYPE), jnp.float32),
            pltpu.VMEM((_GRID, _TN, _NUM_TYPE), jnp.float32),
            pltpu.VMEM((1, _BATCH), jnp.float32),
            pltpu.VMEM((1, _HIST), jnp.float32),
            pltpu.VMEM((1, _HIST, _NUM_TYPE), jnp.int32),
            pltpu.SemaphoreType.DMA((_GRID,)),
            pltpu.SemaphoreType.DMA((3,)),
        ],
    )(ti_row, tjs, cjs, emb_weight)
    return jnp.reshape(out, (_NUM_TYPE, _BATCH, 1))


# speedup vs baseline: 1.0629x; 1.0629x over previous
import jax
import jax.numpy as jnp
from jax.experimental import pallas as pl
from jax.experimental.pallas import tpu as pltpu

_NUM_TYPE = 1000
_BATCH = 1024
_HIST = 200
_TN = 200
_CHUNKS = _NUM_TYPE // _TN
_LANES = 128
_SUB = _BATCH // _LANES


def _body(ti_hbm, tjs_hbm, h_hbm, emb_hbm, out_hbm,
          gtT_ref, hf_ref, ebuf_ref, obuf_ref, ti_ref, tjs_ref, h_ref,
          esems, osems, ssems):
    for blk in range(_CHUNKS):
        pltpu.make_async_copy(
            emb_hbm.at[pl.ds(blk * _TN, _TN), :], ebuf_ref.at[blk],
            esems.at[blk]).start()
    cp_ti = pltpu.make_async_copy(ti_hbm, ti_ref, ssems.at[0])
    cp_tjs = pltpu.make_async_copy(tjs_hbm, tjs_ref, ssems.at[1])
    cp_h = pltpu.make_async_copy(h_hbm, h_ref, ssems.at[2])
    cp_ti.start()
    cp_tjs.start()
    cp_h.start()
    cp_ti.wait()
    cp_tjs.wait()
    gtT_ref[:] = jnp.exp(tjs_ref[0, :][:, None] - ti_ref[0, :][None, :])
    cp_h.wait()
    hf_ref[:] = h_ref[0].astype(jnp.float32)

    for blk in range(_CHUNKS):
        pltpu.make_async_copy(
            emb_hbm.at[pl.ds(blk * _TN, _TN), :], ebuf_ref.at[blk],
            esems.at[blk]).wait()
        P = jax.lax.dot_general(
            ebuf_ref[blk], hf_ref[:], (((1,), (1,)), ((), ())),
            preferred_element_type=jnp.float32)
        res = jnp.dot(P, gtT_ref[:], preferred_element_type=jnp.float32)
        obuf_ref[blk] = res.reshape(_TN, _SUB, _LANES)
        pltpu.make_async_copy(
            obuf_ref.at[blk], out_hbm.at[pl.ds(blk * _TN, _TN)],
            osems.at[blk]).start()

    for blk in range(_CHUNKS):
        pltpu.make_async_copy(
            obuf_ref.at[blk], out_hbm.at[pl.ds(blk * _TN, _TN)],
            osems.at[blk]).wait()


def kernel(ti, tjs, ci, cjs, emb_weight):
    del ci
    ti_row = jnp.reshape(ti, (1, _BATCH))
    out = pl.pallas_call(
        _body,
        in_specs=[
            pl.BlockSpec(memory_space=pl.ANY),
            pl.BlockSpec(memory_space=pl.ANY),
            pl.BlockSpec(memory_space=pl.ANY),
            pl.BlockSpec(memory_space=pl.ANY),
        ],
        out_specs=pl.BlockSpec(memory_space=pl.ANY),
        out_shape=jax.ShapeDtypeStruct((_NUM_TYPE, _SUB, _LANES), jnp.float32),
        scratch_shapes=[
            pltpu.VMEM((_HIST, _BATCH), jnp.float32),
            pltpu.VMEM((_HIST, _NUM_TYPE), jnp.float32),
            pltpu.VMEM((_CHUNKS, _TN, _NUM_TYPE), jnp.float32),
            pltpu.VMEM((_CHUNKS, _TN, _SUB, _LANES), jnp.float32),
            pltpu.VMEM((1, _BATCH), jnp.float32),
            pltpu.VMEM((1, _HIST), jnp.float32),
            pltpu.VMEM((1, _HIST, _NUM_TYPE), jnp.int32),
            pltpu.SemaphoreType.DMA((_CHUNKS,)),
            pltpu.SemaphoreType.DMA((_CHUNKS,)),
            pltpu.SemaphoreType.DMA((3,)),
        ],
    )(ti_row, tjs, cjs, emb_weight)
    return jnp.reshape(out, (_NUM_TYPE, _BATCH, 1))


# speedup vs baseline: 1.0938x; 1.0291x over previous
import jax
import jax.numpy as jnp
from jax.experimental import pallas as pl
from jax.experimental.pallas import tpu as pltpu

_NUM_TYPE = 1000
_BATCH = 1024
_HIST = 200
_TN = 200
_CHUNKS = _NUM_TYPE // _TN
_LANES = 128
_SUB = _BATCH // _LANES


def _body(ti_hbm, tjs_hbm, h_hbm, emb_hbm, out_hbm,
          gtT_ref, hf_ref, ebuf_ref, obuf_ref, ti_ref, tjs_ref, h_ref,
          esems, osems, ssems):
    cp_ti = pltpu.make_async_copy(ti_hbm, ti_ref, ssems.at[0])
    cp_tjs = pltpu.make_async_copy(tjs_hbm, tjs_ref, ssems.at[1])
    cp_h = pltpu.make_async_copy(h_hbm, h_ref, ssems.at[2])
    cp_ti.start()
    cp_tjs.start()
    cp_h.start()
    for blk in range(_CHUNKS):
        pltpu.make_async_copy(
            emb_hbm.at[pl.ds(blk * _TN, _TN), :], ebuf_ref.at[blk],
            esems.at[blk]).start()
    cp_ti.wait()
    cp_tjs.wait()
    gtT_ref[:] = jnp.exp(tjs_ref[0, :][:, None] - ti_ref[0, :][None, :])
    cp_h.wait()
    hf_ref[:] = h_ref[0].astype(jnp.float32)

    for blk in range(_CHUNKS):
        pltpu.make_async_copy(
            emb_hbm.at[pl.ds(blk * _TN, _TN), :], ebuf_ref.at[blk],
            esems.at[blk]).wait()
        P = jax.lax.dot_general(
            ebuf_ref[blk], hf_ref[:], (((1,), (1,)), ((), ())),
            preferred_element_type=jnp.float32)
        res = jnp.dot(P, gtT_ref[:], preferred_element_type=jnp.float32)
        obuf_ref[blk] = res.reshape(_TN, _SUB, _LANES)
        pltpu.make_async_copy(
            obuf_ref.at[blk], out_hbm.at[pl.ds(blk * _TN, _TN)],
            osems.at[blk]).start()

    for blk in range(_CHUNKS):
        pltpu.make_async_copy(
            obuf_ref.at[blk], out_hbm.at[pl.ds(blk * _TN, _TN)],
            osems.at[blk]).wait()


def kernel(ti, tjs, ci, cjs, emb_weight):
    del ci
    ti_row = jnp.reshape(ti, (1, _BATCH))
    out = pl.pallas_call(
        _body,
        in_specs=[
            pl.BlockSpec(memory_space=pl.ANY),
            pl.BlockSpec(memory_space=pl.ANY),
            pl.BlockSpec(memory_space=pl.ANY),
            pl.BlockSpec(memory_space=pl.ANY),
        ],
        out_specs=pl.BlockSpec(memory_space=pl.ANY),
        out_shape=jax.ShapeDtypeStruct((_NUM_TYPE, _SUB, _LANES), jnp.float32),
        scratch_shapes=[
            pltpu.VMEM((_HIST, _BATCH), jnp.float32),
            pltpu.VMEM((_HIST, _NUM_TYPE), jnp.float32),
            pltpu.VMEM((_CHUNKS, _TN, _NUM_TYPE), jnp.float32),
            pltpu.VMEM((_CHUNKS, _TN, _SUB, _LANES), jnp.float32),
            pltpu.VMEM((1, _BATCH), jnp.float32),
            pltpu.VMEM((1, _HIST), jnp.float32),
            pltpu.VMEM((1, _HIST, _NUM_TYPE), jnp.int32),
            pltpu.SemaphoreType.DMA((_CHUNKS,)),
            pltpu.SemaphoreType.DMA((_CHUNKS,)),
            pltpu.SemaphoreType.DMA((3,)),
        ],
    )(ti_row, tjs, cjs, emb_weight)
    return jnp.reshape(out, (_NUM_TYPE, _BATCH, 1))
